# baseline (speedup 1.0000x reference)
import functools

import jax
import jax.numpy as jnp
from jax import lax
from jax.experimental import pallas as pl
from jax.experimental.pallas import tpu as pltpu
from jax.experimental.pallas import tpu_sc as plsc

_N = 1000000
_CLS = 20
_NC = 2
_NS = 16
_NW = _NC * _NS
_ROWS_W = _N // _NW
_ROWS_W_PAD = ((_ROWS_W + 15) // 16) * 16
_P1_CHUNK = 2048
_L = 16
_GB = 128
_TCB = 32768

_SC_PARAMS = pltpu.CompilerParams(
    needs_layout_passes=False, use_tc_tiling_on_sc=False)


@functools.cache
def _mesh():
    return plsc.VectorSubcoreMesh(
        core_axis_name="c", subcore_axis_name="s",
        num_cores=_NC, num_subcores=_NS,
    )


def _wid():
    return lax.axis_index("c") * _NS + lax.axis_index("s")


def _any_lane(m):
    return plsc.all_reduce_population_count(m)[0] > 0


def _apply_body(p_ref, u_ref, w_ref, b_ref, out_ref):
    x = p_ref[...]
    u = u_ref[...]
    ci = lax.broadcasted_iota(jnp.int32, x.shape, 0).astype(jnp.float32)
    c_old = jnp.sum(x * ci, axis=0)
    wb = w_ref[...] + b_ref[...]
    tv = jnp.argmax(wb, axis=1).astype(jnp.float32)
    tcls = jnp.zeros_like(c_old)
    for k in range(_CLS):
        tcls = jnp.where(c_old == k, tv[k], tcls)
    eff = jnp.where(u == 30.0, tcls, u)
    effb = jnp.broadcast_to(eff[None, :], x.shape)
    keep = jnp.broadcast_to((u < 0.0)[None, :], x.shape)
    out_ref[...] = jnp.where(keep, x, (ci == effb).astype(jnp.float32))


def _tc_apply(primT, upd, w, b2d):
    nblk = (_N + _TCB - 1) // _TCB
    return pl.pallas_call(
        _apply_body,
        grid=(nblk,),
        in_specs=[
            pl.BlockSpec((_CLS, _TCB), lambda i: (0, i)),
            pl.BlockSpec((_TCB,), lambda i: (i,)),
            pl.BlockSpec((_CLS, _CLS), lambda i: (0, 0)),
            pl.BlockSpec((1, _CLS), lambda i: (0, 0)),
        ],
        out_specs=pl.BlockSpec((_CLS, _TCB), lambda i: (0, i)),
        out_shape=jax.ShapeDtypeStruct((_CLS, _N), jnp.float32),
    )(primT, upd, w, b2d)


def _p1_body(pos_hbm, val_hbm, bits_hbm, bits_v, pos_v, val_v, sem_a, sem_b):
    npairs = pos_hbm.shape[0]
    nchunk = npairs // _P1_CHUNK
    wid = _wid()
    base = wid * _ROWS_W

    def _zero(i, _):
        bits_v[pl.ds(i * _L, _L)] = jnp.zeros((_L,), jnp.int32)
        return 0

    lax.fori_loop(0, _ROWS_W_PAD // _L, _zero, 0)

    def _dmas(k, sem):
        par = (k % 2) * _P1_CHUNK
        sl = pl.ds(k * _P1_CHUNK, _P1_CHUNK)
        dsl = pl.ds(par, _P1_CHUNK)
        return (pltpu.make_async_copy(pos_hbm.at[sl], pos_v.at[dsl], sem),
                pltpu.make_async_copy(val_hbm.at[sl], val_v.at[dsl], sem))

    def _start(k):
        @pl.when(k % 2 == 0)
        def _():
            for d in _dmas(k, sem_a):
                d.start()

        @pl.when(k % 2 == 1)
        def _():
            for d in _dmas(k, sem_b):
                d.start()

    def _wait(k):
        @pl.when(k % 2 == 0)
        def _():
            for d in _dmas(k, sem_a):
                d.wait()

        @pl.when(k % 2 == 1)
        def _():
            for d in _dmas(k, sem_b):
                d.wait()

    _start(0)

    def _chunk(k, _):
        @pl.when(k + 1 < nchunk)
        def _():
            _start(k + 1)

        _wait(k)
        par = (k % 2) * _P1_CHUNK

        def _group(g, _):
            off = par + g * _L
            pos = pos_v[pl.ds(off, _L)]
            val = val_v[pl.ds(off, _L)]
            rel = pos - base
            m = (rel >= 0) & (rel < _ROWS_W)
            any_in = _any_lane(m)

            @pl.when(any_in)
            def _():
                srel = jnp.clip(rel, 0, _ROWS_W - 1)
                bitv = jnp.int32(1) << val

                def _cond(pending):
                    return _any_lane(pending)

                def _rmw(pending):
                    old = plsc.load_gather(bits_v, [srel], mask=pending)
                    plsc.store_scatter(bits_v, [srel], old | bitv, mask=pending)
                    cur = plsc.load_gather(bits_v, [srel], mask=pending)
                    return pending & ((cur & bitv) != bitv)

                lax.while_loop(_cond, _rmw, m)

            return 0

        lax.fori_loop(0, _P1_CHUNK // _L, _group, 0)
        return 0

    lax.fori_loop(0, nchunk, _chunk, 0)
    pltpu.async_copy(bits_v.at[pl.ds(0, _ROWS_W)], bits_hbm.at[wid], sem_a).wait()


def _sc_phase1(pos1, val1):
    return pl.kernel(
        _p1_body,
        out_type=jax.ShapeDtypeStruct((_NW, _ROWS_W), jnp.int32),
        mesh=_mesh(),
        compiler_params=_SC_PARAMS,
        scratch_types=[
            pltpu.VMEM((_ROWS_W_PAD,), jnp.int32),
            pltpu.VMEM((2 * _P1_CHUNK,), jnp.int32),
            pltpu.VMEM((2 * _P1_CHUNK,), jnp.int32),
            pltpu.SemaphoreType.DMA,
            pltpu.SemaphoreType.DMA,
        ],
    )(pos1, val1)


def _p2_body(upd_ref, bits_hbm, pos2_hbm, w_hbm, b_hbm,
             p2_v, bits_b, av, vv, w_v, b_v, wb_v, t_v,
             sem1, sem2, sem3):
    nrow = p2_v.shape[0]
    wid = _wid()
    pltpu.async_copy(pos2_hbm.at[wid], p2_v, sem1).wait()
    pltpu.async_copy(w_hbm, w_v, sem1).wait()
    pltpu.async_copy(b_hbm, b_v, sem1).wait()

    lanes = lax.iota(jnp.int32, _L)

    def _wb(g, _):
        bidx = (g % 2) * _L + lanes
        wb_v[pl.ds(g * _L, _L)] = (
            w_v[pl.ds(g * _L, _L)] + plsc.load_gather(b_v, [bidx]))
        return 0

    lax.fori_loop(0, 64, _wb, 0)

    for chunk in range(2):
        cvec = chunk * _L + lanes

        def _scan(j, carry):
            best, bi = carry
            lg = plsc.load_gather(wb_v, [cvec * 32 + j])
            take = lg > best
            return jnp.where(take, lg, best), jnp.where(take, j, bi)

        best0 = plsc.load_gather(wb_v, [cvec * 32])
        _, bi = lax.fori_loop(1, _CLS, _scan, (best0, jnp.zeros((_L,), jnp.int32)))
        t_v[pl.ds(chunk * _L, _L)] = bi

    gathers = []
    for j in range(nrow):
        gathers.append(pltpu.async_copy(bits_hbm.at[p2_v.at[j]], bits_b.at[j], sem1))
    for g in gathers:
        g.wait()

    def _row(j, _):
        for l in range(_GB // _L):
            sl = pl.ds(l * _L, _L)
            pos = p2_v[j, sl]
            bits = bits_b[j, sl]
            is0 = bits == 0
            one_hot = (bits & (bits - 1)) == 0
            f = bits.astype(jnp.float32)
            expo = (plsc.bitcast(f, jnp.int32) >> 23) - 127
            c_one = jnp.clip(expo, 0, _CLS - 1)
            cls = plsc.load_gather(t_v, [c_one])

            multi = ~one_hot
            any_multi = _any_lane(multi)

            def _hard():
                def _outer(j2, carry):
                    best, bi = carry

                    def _inner(v, acc):
                        on = ((bits >> v) & 1) == 1
                        wv = plsc.load_gather(
                            w_v, [jnp.broadcast_to(v * 32 + j2, (_L,))])
                        return acc + jnp.where(on, wv, 0.0)

                    lg = lax.fori_loop(0, _CLS, _inner, jnp.zeros((_L,), jnp.float32))
                    lg = lg + plsc.load_gather(b_v, [jnp.broadcast_to(j2, (_L,))])
                    take = lg > best
                    return jnp.where(take, lg, best), jnp.where(take, j2, bi)

                init = (jnp.full((_L,), -jnp.inf, jnp.float32),
                        jnp.zeros((_L,), jnp.int32))
                _, bi = lax.fori_loop(0, _CLS, _outer, init)
                return jnp.where(multi, bi, cls)

            cls = lax.cond(any_multi, _hard, lambda: cls)

            av[j, sl] = pos
            vv[j, sl] = jnp.where(is0, jnp.float32(30.0), cls.astype(jnp.float32))
        return 0

    lax.fori_loop(0, nrow, _row, 0)

    scatters = []
    for j in range(nrow):
        scatters.append(pltpu.async_copy(vv.at[j], upd_ref.at[av.at[j]], sem3))
    for s in scatters:
        s.wait()


def _sc_phase2(upd_flat_ref, bits, pos2, w, b_pad, nrow):
    pl.kernel(
        _p2_body,
        out_type=(),
        mesh=_mesh(),
        compiler_params=_SC_PARAMS,
        scratch_types=[
            pltpu.VMEM((nrow, _GB), jnp.int32),
            pltpu.VMEM((nrow, _GB), jnp.int32),
            pltpu.VMEM((nrow, _GB), jnp.int32),
            pltpu.VMEM((nrow, _GB), jnp.float32),
            pltpu.VMEM((1024,), jnp.float32),
            pltpu.VMEM((32,), jnp.float32),
            pltpu.VMEM((1024,), jnp.float32),
            pltpu.VMEM((32,), jnp.int32),
            pltpu.SemaphoreType.DMA,
            pltpu.SemaphoreType.DMA,
            pltpu.SemaphoreType.DMA,
        ],
    )(upd_flat_ref, bits, pos2, w, b_pad)


def kernel(primary, W, b, positions1, values1, positions2):
    p1 = positions1.shape[0]
    p2 = positions2.shape[0]

    pad1 = (-p1) % _P1_CHUNK
    if pad1:
        positions1 = jnp.concatenate(
            [positions1, jnp.full((pad1,), -1, positions1.dtype)])
        values1 = jnp.concatenate([values1, jnp.zeros((pad1,), values1.dtype)])

    pad2 = (-p2) % (_NW * _GB)
    if pad2:
        reps = -(-pad2 // p2)
        extra = jnp.tile(positions2, reps)[:pad2]
        positions2 = jnp.concatenate([positions2, extra])
    nrow = positions2.shape[0] // (_NW * _GB)
    pos2 = positions2.reshape(_NW, nrow, _GB).astype(jnp.int32)

    b_pad = jnp.concatenate([b, jnp.zeros((32 - _CLS,), b.dtype)])
    w_pad = jnp.zeros((32, 32), W.dtype).at[:_CLS, :_CLS].set(W).reshape(1024)

    primT = primary.T
    upd0 = jnp.full((_N,), -1.0, jnp.float32)
    bits = _sc_phase1(positions1.astype(jnp.int32),
                      values1.astype(jnp.int32)).reshape(_N)

    uref = jax.new_ref(upd0)
    _sc_phase2(uref, bits, pos2, w_pad, b_pad, nrow)
    outT = _tc_apply(primT, uref[...], W, b.reshape(1, _CLS))
    return outT.T


# speedup vs baseline: 1.3971x; 1.3971x over previous
import functools

import jax
import jax.numpy as jnp
from jax import lax
from jax.experimental import pallas as pl
from jax.experimental.pallas import tpu as pltpu
from jax.experimental.pallas import tpu_sc as plsc

_N = 1000000
_CLS = 20
_NC = 2
_NS = 16
_NW = _NC * _NS
_ROWS_W = _N // _NW
_ROWS_W_PAD = ((_ROWS_W + 15) // 16) * 16
_P1_CHUNK = 2048
_L = 16
_GB = 128
_TCB = 32768

_SC_PARAMS = pltpu.CompilerParams(
    needs_layout_passes=False, use_tc_tiling_on_sc=False)


@functools.cache
def _mesh():
    return plsc.VectorSubcoreMesh(
        core_axis_name="c", subcore_axis_name="s",
        num_cores=_NC, num_subcores=_NS,
    )


def _wid():
    return lax.axis_index("c") * _NS + lax.axis_index("s")


def _any_lane(m):
    return plsc.all_reduce_population_count(m)[0] > 0


def _prep_body(p_ref, cls_ref, upd_ref):
    x = p_ref[...]
    ci = lax.broadcasted_iota(jnp.int32, x.shape, 0).astype(jnp.float32)
    cls_ref[...] = jnp.sum(x * ci, axis=0)
    upd_ref[...] = jnp.full(upd_ref.shape, -1.0, jnp.float32)


def _tc_prep(primT):
    nblk = (_N + _TCB - 1) // _TCB
    return pl.pallas_call(
        _prep_body,
        grid=(nblk,),
        in_specs=[pl.BlockSpec((_CLS, _TCB), lambda i: (0, i))],
        out_specs=[
            pl.BlockSpec((_TCB,), lambda i: (i,)),
            pl.BlockSpec((_TCB,), lambda i: (i,)),
        ],
        out_shape=[
            jax.ShapeDtypeStruct((_N,), jnp.float32),
            jax.ShapeDtypeStruct((_N,), jnp.float32),
        ],
    )(primT)


def _apply_body(p_ref, u_ref, out_ref):
    x = p_ref[...]
    u = u_ref[...]
    ub = jnp.broadcast_to(u[None, :], x.shape)
    ci = lax.broadcasted_iota(jnp.int32, x.shape, 0).astype(jnp.float32)
    out_ref[...] = jnp.where(ub < 0.0, x, (ci == ub).astype(jnp.float32))


def _tc_apply(primT, upd):
    nblk = (_N + _TCB - 1) // _TCB
    return pl.pallas_call(
        _apply_body,
        grid=(nblk,),
        in_specs=[
            pl.BlockSpec((_CLS, _TCB), lambda i: (0, i)),
            pl.BlockSpec((_TCB,), lambda i: (i,)),
        ],
        out_specs=pl.BlockSpec((_CLS, _TCB), lambda i: (0, i)),
        out_shape=jax.ShapeDtypeStruct((_CLS, _N), jnp.float32),
    )(primT, upd)


def _p1_body(pos_hbm, val_hbm, bits_hbm, bits_v, pos_v, val_v, sem_a, sem_b):
    npairs = pos_hbm.shape[0]
    nchunk = npairs // _P1_CHUNK
    ngrp = _P1_CHUNK // _L
    wid = _wid()
    base = wid * _ROWS_W

    def _zero(i, _):
        bits_v[pl.ds(i * _L, _L)] = jnp.zeros((_L,), jnp.int32)
        return 0

    lax.fori_loop(0, _ROWS_W_PAD // _L, _zero, 0)

    def _dmas(k, sem):
        par = (k % 2) * _P1_CHUNK
        sl = pl.ds(k * _P1_CHUNK, _P1_CHUNK)
        dsl = pl.ds(par, _P1_CHUNK)
        return (pltpu.make_async_copy(pos_hbm.at[sl], pos_v.at[dsl], sem),
                pltpu.make_async_copy(val_hbm.at[sl], val_v.at[dsl], sem))

    def _start(k):
        @pl.when(k % 2 == 0)
        def _():
            for d in _dmas(k, sem_a):
                d.start()

        @pl.when(k % 2 == 1)
        def _():
            for d in _dmas(k, sem_b):
                d.start()

    def _wait(k):
        @pl.when(k % 2 == 0)
        def _():
            for d in _dmas(k, sem_a):
                d.wait()

        @pl.when(k % 2 == 1)
        def _():
            for d in _dmas(k, sem_b):
                d.wait()

    _start(0)

    def _decode(par, g):
        off = par + g * _L
        pos = pos_v[pl.ds(off, _L)]
        val = val_v[pl.ds(off, _L)]
        rel = pos - base
        m = (rel >= 0) & (rel < _ROWS_W)
        srel = jnp.clip(rel, 0, _ROWS_W - 1)
        bitv = jnp.int32(1) << val
        return m, srel, bitv

    def _chunk(k, _):
        @pl.when(k + 1 < nchunk)
        def _():
            _start(k + 1)

        _wait(k)
        par = (k % 2) * _P1_CHUNK

        def _g1(g, _):
            m, srel, bitv = _decode(par, g)
            old = plsc.load_gather(bits_v, [srel], mask=m)
            plsc.store_scatter(bits_v, [srel], old | bitv, mask=m)
            return 0

        lax.fori_loop(0, ngrp, _g1, 0)
        pl.delay(64)

        def _g2(g, failv):
            m, srel, bitv = _decode(par, g)
            cur = plsc.load_gather(bits_v, [srel], mask=m)
            return failv | (m & ((cur & bitv) != bitv))

        failv = lax.fori_loop(0, ngrp, _g2, jnp.zeros((_L,), jnp.bool_))

        @pl.when(_any_lane(failv))
        def _():
            def _g3(g, _):
                m, srel, bitv = _decode(par, g)

                def _cond(pending):
                    return _any_lane(pending)

                def _rmw(pending):
                    old = plsc.load_gather(bits_v, [srel], mask=pending)
                    plsc.store_scatter(bits_v, [srel], old | bitv, mask=pending)
                    cur = plsc.load_gather(bits_v, [srel], mask=pending)
                    return pending & ((cur & bitv) != bitv)

                lax.while_loop(_cond, _rmw, m)
                return 0

            lax.fori_loop(0, ngrp, _g3, 0)

        return 0

    lax.fori_loop(0, nchunk, _chunk, 0)
    pltpu.async_copy(bits_v.at[pl.ds(0, _ROWS_W)], bits_hbm.at[wid], sem_a).wait()


def _sc_phase1(pos1, val1):
    return pl.kernel(
        _p1_body,
        out_type=jax.ShapeDtypeStruct((_NW, _ROWS_W), jnp.int32),
        mesh=_mesh(),
        compiler_params=_SC_PARAMS,
        scratch_types=[
            pltpu.VMEM((_ROWS_W_PAD,), jnp.int32),
            pltpu.VMEM((2 * _P1_CHUNK,), jnp.int32),
            pltpu.VMEM((2 * _P1_CHUNK,), jnp.int32),
            pltpu.SemaphoreType.DMA,
            pltpu.SemaphoreType.DMA,
        ],
    )(pos1, val1)


def _p2_body(upd_ref, bits_hbm, clsp_hbm, pos2_hbm, w_hbm, b_hbm,
             p2_v, bits_b, cls_b, av, vv, w_v, b_v, wb_v, t_v,
             sem1, sem2, sem3):
    nrow = p2_v.shape[0]
    wid = _wid()
    pltpu.async_copy(pos2_hbm.at[wid], p2_v, sem1).wait()
    pltpu.async_copy(w_hbm, w_v, sem1).wait()
    pltpu.async_copy(b_hbm, b_v, sem1).wait()

    lanes = lax.iota(jnp.int32, _L)

    def _wb(g, _):
        bidx = (g % 2) * _L + lanes
        wb_v[pl.ds(g * _L, _L)] = (
            w_v[pl.ds(g * _L, _L)] + plsc.load_gather(b_v, [bidx]))
        return 0

    lax.fori_loop(0, 64, _wb, 0)

    for chunk in range(2):
        cvec = chunk * _L + lanes

        def _scan(j, carry):
            best, bi = carry
            lg = plsc.load_gather(wb_v, [cvec * 32 + j])
            take = lg > best
            return jnp.where(take, lg, best), jnp.where(take, j, bi)

        best0 = plsc.load_gather(wb_v, [cvec * 32])
        _, bi = lax.fori_loop(1, _CLS, _scan, (best0, jnp.zeros((_L,), jnp.int32)))
        t_v[pl.ds(chunk * _L, _L)] = bi

    gathers = []
    for j in range(nrow):
        gathers.append(pltpu.async_copy(bits_hbm.at[p2_v.at[j]], bits_b.at[j], sem1))
        gathers.append(pltpu.async_copy(clsp_hbm.at[p2_v.at[j]], cls_b.at[j], sem2))
    for g in gathers:
        g.wait()

    def _row(j, _):
        for l in range(_GB // _L):
            sl = pl.ds(l * _L, _L)
            pos = p2_v[j, sl]
            bits = bits_b[j, sl]
            c_old = cls_b[j, sl].astype(jnp.int32)
            is0 = bits == 0
            one_hot = (bits & (bits - 1)) == 0
            f = bits.astype(jnp.float32)
            expo = (plsc.bitcast(f, jnp.int32) >> 23) - 127
            c_one = jnp.clip(jnp.where(is0, c_old, expo), 0, _CLS - 1)
            cls = plsc.load_gather(t_v, [c_one])

            multi = ~one_hot
            any_multi = _any_lane(multi)

            def _hard():
                def _outer(j2, carry):
                    best, bi = carry

                    def _inner(v, acc):
                        on = ((bits >> v) & 1) == 1
                        wv = plsc.load_gather(
                            w_v, [jnp.broadcast_to(v * 32 + j2, (_L,))])
                        return acc + jnp.where(on, wv, 0.0)

                    lg = lax.fori_loop(0, _CLS, _inner, jnp.zeros((_L,), jnp.float32))
                    lg = lg + plsc.load_gather(b_v, [jnp.broadcast_to(j2, (_L,))])
                    take = lg > best
                    return jnp.where(take, lg, best), jnp.where(take, j2, bi)

                init = (jnp.full((_L,), -jnp.inf, jnp.float32),
                        jnp.zeros((_L,), jnp.int32))
                _, bi = lax.fori_loop(0, _CLS, _outer, init)
                return jnp.where(multi, bi, cls)

            cls = lax.cond(any_multi, _hard, lambda: cls)

            av[j, sl] = pos
            vv[j, sl] = cls.astype(jnp.float32)
        return 0

    lax.fori_loop(0, nrow, _row, 0)

    scatters = []
    for j in range(nrow):
        scatters.append(pltpu.async_copy(vv.at[j], upd_ref.at[av.at[j]], sem3))
    for s in scatters:
        s.wait()


def _sc_phase2(upd_flat_ref, bits, clsp, pos2, w, b_pad, nrow):
    pl.kernel(
        _p2_body,
        out_type=(),
        mesh=_mesh(),
        compiler_params=_SC_PARAMS,
        scratch_types=[
            pltpu.VMEM((nrow, _GB), jnp.int32),
            pltpu.VMEM((nrow, _GB), jnp.int32),
            pltpu.VMEM((nrow, _GB), jnp.float32),
            pltpu.VMEM((nrow, _GB), jnp.int32),
            pltpu.VMEM((nrow, _GB), jnp.float32),
            pltpu.VMEM((1024,), jnp.float32),
            pltpu.VMEM((32,), jnp.float32),
            pltpu.VMEM((1024,), jnp.float32),
            pltpu.VMEM((32,), jnp.int32),
            pltpu.SemaphoreType.DMA,
            pltpu.SemaphoreType.DMA,
            pltpu.SemaphoreType.DMA,
        ],
    )(upd_flat_ref, bits, clsp, pos2, w, b_pad)


def kernel(primary, W, b, positions1, values1, positions2):
    p1 = positions1.shape[0]
    p2 = positions2.shape[0]

    pad1 = (-p1) % _P1_CHUNK
    if pad1:
        positions1 = jnp.concatenate(
            [positions1, jnp.full((pad1,), -1, positions1.dtype)])
        values1 = jnp.concatenate([values1, jnp.zeros((pad1,), values1.dtype)])

    pad2 = (-p2) % (_NW * _GB)
    if pad2:
        reps = -(-pad2 // p2)
        extra = jnp.tile(positions2, reps)[:pad2]
        positions2 = jnp.concatenate([positions2, extra])
    nrow = positions2.shape[0] // (_NW * _GB)
    pos2 = positions2.reshape(_NW, nrow, _GB).astype(jnp.int32)

    b_pad = jnp.concatenate([b, jnp.zeros((32 - _CLS,), b.dtype)])
    w_pad = jnp.zeros((32, 32), W.dtype).at[:_CLS, :_CLS].set(W).reshape(1024)

    primT = primary.T
    clsp, upd0 = _tc_prep(primT)
    bits = _sc_phase1(positions1.astype(jnp.int32),
                      values1.astype(jnp.int32)).reshape(_N)

    uref = jax.new_ref(upd0)
    _sc_phase2(uref, bits, clsp, pos2, w_pad, b_pad, nrow)
    outT = _tc_apply(primT, uref[...])
    return outT.T
